# baseline (speedup 1.0000x reference)
import functools

import jax
import jax.numpy as jnp
from jax import lax
from jax.experimental import pallas as pl
from jax.experimental.pallas import tpu as pltpu
from jax.experimental.pallas import tpu_sc as plsc

B = 16384
L = 50
LP = 56
D = 64
NC, NS = 2, 16
NW = NC * NS
RPW = B // NW
INV_L = 1.0 / L


def _pool_body(usr_hbm, desc_hbm, tum, tim, tug, tig,
               oum, oim, oug, oig,
               idx_u, idx_d, buf0, buf1, out_v, sem0, sem1):
  wid = lax.axis_index("s") * NC + lax.axis_index("c")
  base = wid * RPW
  pltpu.sync_copy(usr_hbm.at[pl.ds(base, RPW)], idx_u)
  pltpu.sync_copy(desc_hbm.at[pl.ds(base, RPW)], idx_d)

  bufs = (buf0, buf1)
  sems = (sem0, sem1)

  def run_pass(idx_v, table, out_hbm):
    def issue(b, k):
      pltpu.async_copy(table.at[idx_v.at[b, pl.ds(0, LP)]], bufs[k], sems[k])

    def wait(k):
      pltpu.make_async_copy(
          table.at[idx_v.at[0, pl.ds(0, LP)]], bufs[k], sems[k]).wait()

    def acc_and_store(b, k):
      buf = bufs[k]
      zero = jnp.zeros((16,), jnp.float32)

      def rbody(r, carry):
        return tuple(carry[j] + buf[r, pl.ds(16 * j, 16)] for j in range(4))

      a = lax.fori_loop(0, L, rbody, (zero, zero, zero, zero), unroll=2)
      for j in range(4):
        out_v[b, pl.ds(16 * j, 16)] = a[j] * INV_L

    issue(0, 0)

    def pair_body(p, carry):
      b = p * 2
      issue(b + 1, 1)
      wait(0)
      acc_and_store(b, 0)

      @pl.when(b + 2 < RPW)
      def _():
        issue(b + 2, 0)

      wait(1)
      acc_and_store(b + 1, 1)
      return carry

    lax.fori_loop(0, RPW // 2, pair_body, 0)
    pltpu.sync_copy(out_v, out_hbm.at[pl.ds(base, RPW)])

  run_pass(idx_u, tum, oum)
  run_pass(idx_u, tug, oug)
  run_pass(idx_d, tim, oim)
  run_pass(idx_d, tig, oig)


_pool = functools.partial(
    pl.kernel,
    out_type=[jax.ShapeDtypeStruct((B, D), jnp.float32)] * 4,
    mesh=plsc.VectorSubcoreMesh(
        core_axis_name="c", subcore_axis_name="s",
        num_cores=NC, num_subcores=NS),
    compiler_params=pltpu.CompilerParams(use_tc_tiling_on_sc=False),
    scratch_types=[
        pltpu.VMEM((RPW, LP), jnp.int32),
        pltpu.VMEM((RPW, LP), jnp.int32),
        pltpu.VMEM((LP, D), jnp.float32),
        pltpu.VMEM((LP, D), jnp.float32),
        pltpu.VMEM((RPW, D), jnp.float32),
        pltpu.SemaphoreType.DMA,
        pltpu.SemaphoreType.DMA,
    ],
)(_pool_body)


BB = 2048


def _mlp_body(um_ref, im_ref, ug_ref, ig_ref,
              W1_ref, b1_ref, W2_ref, b2_ref, Wa_ref, ba_ref, out_ref):
  dn = (((1,), (1,)), ((), ()))
  f32 = jnp.float32
  um = um_ref[...]
  im = im_ref[...]
  W1 = W1_ref[...]
  h = (lax.dot_general(um, W1[:, :D], dn, preferred_element_type=f32)
       + lax.dot_general(im, W1[:, D:], dn, preferred_element_type=f32)
       + b1_ref[...])
  h = jnp.maximum(h, 0.0)
  h = lax.dot_general(h, W2_ref[...], dn, preferred_element_type=f32) + b2_ref[...]
  h = jnp.maximum(h, 0.0)
  g = ug_ref[...] * ig_ref[...]
  Wa = Wa_ref[...]
  out = (lax.dot_general(h, Wa[:, :32], dn, preferred_element_type=f32)
         + lax.dot_general(g, Wa[:, 32:], dn, preferred_element_type=f32)
         + ba_ref[...])
  out_ref[...] = out


_mlp = pl.pallas_call(
    _mlp_body,
    grid=(B // BB,),
    in_specs=[
        pl.BlockSpec((BB, D), lambda i: (i, 0)),
        pl.BlockSpec((BB, D), lambda i: (i, 0)),
        pl.BlockSpec((BB, D), lambda i: (i, 0)),
        pl.BlockSpec((BB, D), lambda i: (i, 0)),
        pl.BlockSpec((64, 128), lambda i: (0, 0)),
        pl.BlockSpec((1, 64), lambda i: (0, 0)),
        pl.BlockSpec((32, 64), lambda i: (0, 0)),
        pl.BlockSpec((1, 32), lambda i: (0, 0)),
        pl.BlockSpec((1, 96), lambda i: (0, 0)),
        pl.BlockSpec((1, 1), lambda i: (0, 0)),
    ],
    out_specs=pl.BlockSpec((BB, 1), lambda i: (i, 0)),
    out_shape=jax.ShapeDtypeStruct((B, 1), jnp.float32),
)


def kernel(usr_comments, descriptions, emb_user_mlp, emb_item_mlp,
           emb_user_gmf, emb_item_gmf, W1, b1, W2, b2, Wa, ba):
  pad = ((0, 0), (0, LP - L))
  usr_p = jnp.pad(usr_comments, pad)
  desc_p = jnp.pad(descriptions, pad)
  um, im, ug, ig = _pool(usr_p, desc_p, emb_user_mlp, emb_item_mlp,
                         emb_user_gmf, emb_item_gmf)
  return _mlp(um, im, ug, ig, W1, b1.reshape(1, -1), W2, b2.reshape(1, -1),
              Wa, ba.reshape(1, 1))


# speedup vs baseline: 1.0212x; 1.0212x over previous
import functools

import jax
import jax.numpy as jnp
from jax import lax
from jax.experimental import pallas as pl
from jax.experimental.pallas import tpu as pltpu
from jax.experimental.pallas import tpu_sc as plsc

B = 16384
L = 50
LP = 56
D = 64
NC, NS = 2, 16
NW = NC * NS
RPW = B // NW
INV_L = 1.0 / L

CPB = 2
CH_I = CPB * LP
NCH = RPW // CPB
NBUF = 8
NGRP = NCH // NBUF


def _pool_body(usr_hbm, desc_hbm, tum, tim, tug, tig,
               oum, oim, oug, oig,
               idx_v, b0, b1, b2, b3, b4, b5, b6, b7,
               out_v, s0, s1, s2, s3, s4, s5, s6, s7):
  wid = lax.axis_index("s") * NC + lax.axis_index("c")
  base = wid * RPW

  bufs = (b0, b1, b2, b3, b4, b5, b6, b7)
  sems = (s0, s1, s2, s3, s4, s5, s6, s7)

  def run_pass(idx_hbm, table, out_hbm):
    pltpu.sync_copy(idx_hbm.at[pl.ds(base * LP, RPW * LP)], idx_v)

    def issue(c, j):
      pltpu.async_copy(
          table.at[idx_v.at[pl.ds(c * CH_I, CH_I)]], bufs[j], sems[j])

    def wait(j):
      pltpu.make_async_copy(
          table.at[idx_v.at[pl.ds(0, CH_I)]], bufs[j], sems[j]).wait()

    def acc_chunk(c, j):
      buf = bufs[j]
      zero = jnp.zeros((16,), jnp.float32)

      def rbody(r, carry):
        a = list(carry)
        for q in range(4):
          a[q] = a[q] + buf[r, pl.ds(16 * q, 16)]
        for q in range(4):
          a[4 + q] = a[4 + q] + buf[LP + r, pl.ds(16 * q, 16)]
        return tuple(a)

      a = lax.fori_loop(0, L, rbody, (zero,) * 8, unroll=2)
      row = c * CPB
      for q in range(4):
        out_v[row, pl.ds(16 * q, 16)] = a[q] * INV_L
      for q in range(4):
        out_v[row + 1, pl.ds(16 * q, 16)] = a[4 + q] * INV_L

    for k in range(NBUF):
      issue(k, k)

    def grp_body(g, carry):
      for j in range(NBUF):
        c = g * NBUF + j
        wait(j)
        acc_chunk(c, j)

        @pl.when(c + NBUF < NCH)
        def _():
          issue(c + NBUF, j)

      return carry

    lax.fori_loop(0, NGRP, grp_body, 0)
    pltpu.sync_copy(out_v, out_hbm.at[pl.ds(base, RPW)])

  run_pass(usr_hbm, tum, oum)
  run_pass(usr_hbm, tug, oug)
  run_pass(desc_hbm, tim, oim)
  run_pass(desc_hbm, tig, oig)


_pool = functools.partial(
    pl.kernel,
    out_type=[jax.ShapeDtypeStruct((B, D), jnp.float32)] * 4,
    mesh=plsc.VectorSubcoreMesh(
        core_axis_name="c", subcore_axis_name="s",
        num_cores=NC, num_subcores=NS),
    compiler_params=pltpu.CompilerParams(use_tc_tiling_on_sc=False),
    scratch_types=(
        [pltpu.VMEM((RPW * LP,), jnp.int32)]
        + [pltpu.VMEM((CH_I, D), jnp.float32) for _ in range(NBUF)]
        + [pltpu.VMEM((RPW, D), jnp.float32)]
        + [pltpu.SemaphoreType.DMA for _ in range(NBUF)]
    ),
)(_pool_body)


BB = 2048


def _mlp_body(um_ref, im_ref, ug_ref, ig_ref,
              W1_ref, b1_ref, W2_ref, b2_ref, Wa_ref, ba_ref, out_ref):
  dn = (((1,), (1,)), ((), ()))
  f32 = jnp.float32
  um = um_ref[...]
  im = im_ref[...]
  W1 = W1_ref[...]
  h = (lax.dot_general(um, W1[:, :D], dn, preferred_element_type=f32)
       + lax.dot_general(im, W1[:, D:], dn, preferred_element_type=f32)
       + b1_ref[...])
  h = jnp.maximum(h, 0.0)
  h = lax.dot_general(h, W2_ref[...], dn, preferred_element_type=f32) + b2_ref[...]
  h = jnp.maximum(h, 0.0)
  g = ug_ref[...] * ig_ref[...]
  Wa = Wa_ref[...]
  out = (lax.dot_general(h, Wa[:, :32], dn, preferred_element_type=f32)
         + lax.dot_general(g, Wa[:, 32:], dn, preferred_element_type=f32)
         + ba_ref[...])
  out_ref[...] = out


_mlp = pl.pallas_call(
    _mlp_body,
    grid=(B // BB,),
    in_specs=[
        pl.BlockSpec((BB, D), lambda i: (i, 0)),
        pl.BlockSpec((BB, D), lambda i: (i, 0)),
        pl.BlockSpec((BB, D), lambda i: (i, 0)),
        pl.BlockSpec((BB, D), lambda i: (i, 0)),
        pl.BlockSpec((64, 128), lambda i: (0, 0)),
        pl.BlockSpec((1, 64), lambda i: (0, 0)),
        pl.BlockSpec((32, 64), lambda i: (0, 0)),
        pl.BlockSpec((1, 32), lambda i: (0, 0)),
        pl.BlockSpec((1, 96), lambda i: (0, 0)),
        pl.BlockSpec((1, 1), lambda i: (0, 0)),
    ],
    out_specs=pl.BlockSpec((BB, 1), lambda i: (i, 0)),
    out_shape=jax.ShapeDtypeStruct((B, 1), jnp.float32),
)


def kernel(usr_comments, descriptions, emb_user_mlp, emb_item_mlp,
           emb_user_gmf, emb_item_gmf, W1, b1, W2, b2, Wa, ba):
  pad = ((0, 0), (0, LP - L))
  usr_p = jnp.pad(usr_comments, pad).reshape(-1)
  desc_p = jnp.pad(descriptions, pad).reshape(-1)
  um, im, ug, ig = _pool(usr_p, desc_p, emb_user_mlp, emb_item_mlp,
                         emb_user_gmf, emb_item_gmf)
  return _mlp(um, im, ug, ig, W1, b1.reshape(1, -1), W2, b2.reshape(1, -1),
              Wa, ba.reshape(1, 1))


# speedup vs baseline: 4.0760x; 3.9913x over previous
import functools

import jax
import jax.numpy as jnp
from jax import lax
from jax.experimental import pallas as pl
from jax.experimental.pallas import tpu as pltpu
from jax.experimental.pallas import tpu_sc as plsc

B = 16384
L = 50
V = 1000000
LP = 56
D = 64
NC, NS = 2, 16
NW = NC * NS
RPW = B // NW
INV_L = 1.0 / L

CPB = 2
CH_I = CPB * LP
NCH = RPW // CPB
NBUF = 8
NGRP = NCH // NBUF


def _pool_body(usr_hbm, desc_hbm, tum, tim, tug, tig,
               oum, oim, oug, oig,
               idx_v, b0, b1, b2, b3, b4, b5, b6, b7,
               out_v, s0, s1, s2, s3, s4, s5, s6, s7):
  wid = lax.axis_index("s") * NC + lax.axis_index("c")
  base = wid * RPW

  bufs = (b0, b1, b2, b3, b4, b5, b6, b7)
  sems = (s0, s1, s2, s3, s4, s5, s6, s7)

  def run_pass(idx_hbm, table, out_hbm):
    pltpu.sync_copy(idx_hbm.at[pl.ds(base * LP, RPW * LP)], idx_v)

    def issue(c, j):
      pltpu.async_copy(
          table.at[idx_v.at[pl.ds(c * CH_I, CH_I)]], bufs[j], sems[j])

    def wait(j):
      pltpu.make_async_copy(
          table.at[idx_v.at[pl.ds(0, CH_I)]], bufs[j], sems[j]).wait()

    def acc_chunk(c, j):
      buf = bufs[j]
      zero = jnp.zeros((16,), jnp.float32)

      def rbody(r, carry):
        a = list(carry)
        for q in range(4):
          a[q] = a[q] + buf[r, pl.ds(16 * q, 16)]
        for q in range(4):
          a[4 + q] = a[4 + q] + buf[LP + r, pl.ds(16 * q, 16)]
        return tuple(a)

      a = lax.fori_loop(0, L, rbody, (zero,) * 8, unroll=2)
      row = c * CPB
      for q in range(4):
        out_v[row, pl.ds(16 * q, 16)] = a[q] * INV_L
      for q in range(4):
        out_v[row + 1, pl.ds(16 * q, 16)] = a[4 + q] * INV_L

    for k in range(NBUF):
      issue(k, k)

    def grp_body(g, carry):
      for j in range(NBUF):
        c = g * NBUF + j
        wait(j)
        acc_chunk(c, j)

        @pl.when(c + NBUF < NCH)
        def _():
          issue(c + NBUF, j)

      return carry

    lax.fori_loop(0, NGRP, grp_body, 0)
    pltpu.sync_copy(out_v, out_hbm.at[pl.ds(base, RPW)])

  run_pass(usr_hbm, tum, oum)
  run_pass(usr_hbm, tug, oug)
  run_pass(desc_hbm, tim, oim)
  run_pass(desc_hbm, tig, oig)


_pool = functools.partial(
    pl.kernel,
    out_type=[jax.ShapeDtypeStruct((B, D), jnp.float32)] * 4,
    mesh=plsc.VectorSubcoreMesh(
        core_axis_name="c", subcore_axis_name="s",
        num_cores=NC, num_subcores=NS),
    compiler_params=pltpu.CompilerParams(use_tc_tiling_on_sc=False),
    scratch_types=(
        [pltpu.VMEM((RPW * LP,), jnp.int32)]
        + [pltpu.VMEM((CH_I, D), jnp.float32) for _ in range(NBUF)]
        + [pltpu.VMEM((RPW, D), jnp.float32)]
        + [pltpu.SemaphoreType.DMA for _ in range(NBUF)]
    ),
)(_pool_body)


BB = 2048


def _mlp_body(um_ref, im_ref, ug_ref, ig_ref,
              W1_ref, b1_ref, W2_ref, b2_ref, Wa_ref, ba_ref, out_ref):
  dn = (((1,), (1,)), ((), ()))
  f32 = jnp.float32
  um = um_ref[...]
  im = im_ref[...]
  W1 = W1_ref[...]
  h = (lax.dot_general(um, W1[:, :D], dn, preferred_element_type=f32)
       + lax.dot_general(im, W1[:, D:], dn, preferred_element_type=f32)
       + b1_ref[...])
  h = jnp.maximum(h, 0.0)
  h = lax.dot_general(h, W2_ref[...], dn, preferred_element_type=f32) + b2_ref[...]
  h = jnp.maximum(h, 0.0)
  g = ug_ref[...] * ig_ref[...]
  Wa = Wa_ref[...]
  out = (lax.dot_general(h, Wa[:, :32], dn, preferred_element_type=f32)
         + lax.dot_general(g, Wa[:, 32:], dn, preferred_element_type=f32)
         + ba_ref[...])
  out_ref[...] = out


_mlp = pl.pallas_call(
    _mlp_body,
    grid=(B // BB,),
    in_specs=[
        pl.BlockSpec((BB, D), lambda i: (i, 0)),
        pl.BlockSpec((BB, D), lambda i: (i, 0)),
        pl.BlockSpec((BB, D), lambda i: (i, 0)),
        pl.BlockSpec((BB, D), lambda i: (i, 0)),
        pl.BlockSpec((64, 128), lambda i: (0, 0)),
        pl.BlockSpec((1, 64), lambda i: (0, 0)),
        pl.BlockSpec((32, 64), lambda i: (0, 0)),
        pl.BlockSpec((1, 32), lambda i: (0, 0)),
        pl.BlockSpec((1, 96), lambda i: (0, 0)),
        pl.BlockSpec((1, 1), lambda i: (0, 0)),
    ],
    out_specs=pl.BlockSpec((BB, 1), lambda i: (i, 0)),
    out_shape=jax.ShapeDtypeStruct((B, 1), jnp.float32),
)


def kernel(usr_comments, descriptions, emb_user_mlp, emb_item_mlp,
           emb_user_gmf, emb_item_gmf, W1, b1, W2, b2, Wa, ba):
  pad_vals = jnp.arange(B * (LP - L), dtype=jnp.int32).reshape(B, LP - L) % V
  usr_p = jnp.concatenate([usr_comments, pad_vals], axis=1).reshape(-1)
  desc_p = jnp.concatenate([descriptions, pad_vals], axis=1).reshape(-1)
  um, im, ug, ig = _pool(usr_p, desc_p, emb_user_mlp, emb_item_mlp,
                         emb_user_gmf, emb_item_gmf)
  return _mlp(um, im, ug, ig, W1, b1.reshape(1, -1), W2, b2.reshape(1, -1),
              Wa, ba.reshape(1, 1))


# speedup vs baseline: 7.0895x; 1.7393x over previous
import functools

import jax
import jax.numpy as jnp
from jax import lax
from jax.experimental import pallas as pl
from jax.experimental.pallas import tpu as pltpu
from jax.experimental.pallas import tpu_sc as plsc

B = 16384
L = 50
V = 1000000
LP = 56
D = 64
DP = 2 * D
NC, NS = 2, 16
NW = NC * NS
RPW = B // NW
INV_L = 1.0 / L

NBUF = 8
HALF = RPW // 2
NGRP = HALF // NBUF


CB = 2048


def _fuse_body(au_ref, bu_ref, ai_ref, bi_ref, ou_ref, oi_ref):
  ou_ref[:, :D] = au_ref[...].T
  ou_ref[:, D:] = bu_ref[...].T
  oi_ref[:, :D] = ai_ref[...].T
  oi_ref[:, D:] = bi_ref[...].T


_fuse = pl.pallas_call(
    _fuse_body,
    grid=(V // CB,),
    in_specs=[pl.BlockSpec((D, CB), lambda i: (0, i))] * 4,
    out_specs=[pl.BlockSpec((CB, DP), lambda i: (i, 0))] * 2,
    out_shape=[jax.ShapeDtypeStruct((V, DP), jnp.float32)] * 2,
)


def _pool_body(usr_hbm, desc_hbm, tu, ti, ou, oi,
               idx_v, b0, b1, b2, b3, b4, b5, b6, b7,
               out_v, s0, s1, s2, s3, s4, s5, s6, s7):
  wid = lax.axis_index("s") * NC + lax.axis_index("c")
  base = wid * RPW

  bufs = (b0, b1, b2, b3, b4, b5, b6, b7)
  sems = (s0, s1, s2, s3, s4, s5, s6, s7)

  def run_pass(idx_hbm, table, out_hbm):
    pltpu.sync_copy(idx_hbm.at[pl.ds(base * LP, RPW * LP)], idx_v)

    def issue(c, j):
      pltpu.async_copy(
          table.at[idx_v.at[pl.ds(c * LP, LP)]], bufs[j], sems[j])

    def wait(j):
      pltpu.make_async_copy(
          table.at[idx_v.at[pl.ds(0, LP)]], bufs[j], sems[j]).wait()

    def acc_row(c, row, j):
      buf = bufs[j]
      zero = jnp.zeros((16,), jnp.float32)

      def rbody(r, carry):
        return tuple(carry[q] + buf[r, pl.ds(16 * q, 16)] for q in range(8))

      a = lax.fori_loop(0, L, rbody, (zero,) * 8, unroll=2)
      for q in range(8):
        out_v[row, pl.ds(16 * q, 16)] = a[q] * INV_L

    for h in range(2):
      lo = h * HALF
      for k in range(NBUF):
        issue(lo + k, k)

      def grp_body(g, carry):
        for j in range(NBUF):
          c = lo + g * NBUF + j
          wait(j)
          acc_row(c, c - lo, j)

          @pl.when(c + NBUF < lo + HALF)
          def _():
            issue(c + NBUF, j)

        return carry

      lax.fori_loop(0, NGRP, grp_body, 0)
      pltpu.sync_copy(out_v, out_hbm.at[pl.ds(base + lo, HALF)])

  run_pass(usr_hbm, tu, ou)
  run_pass(desc_hbm, ti, oi)


_pool = functools.partial(
    pl.kernel,
    out_type=[jax.ShapeDtypeStruct((B, DP), jnp.float32)] * 2,
    mesh=plsc.VectorSubcoreMesh(
        core_axis_name="c", subcore_axis_name="s",
        num_cores=NC, num_subcores=NS),
    compiler_params=pltpu.CompilerParams(use_tc_tiling_on_sc=False),
    scratch_types=(
        [pltpu.VMEM((RPW * LP,), jnp.int32)]
        + [pltpu.VMEM((LP, DP), jnp.float32) for _ in range(NBUF)]
        + [pltpu.VMEM((HALF, DP), jnp.float32)]
        + [pltpu.SemaphoreType.DMA for _ in range(NBUF)]
    ),
)(_pool_body)


BB = 2048


def _mlp_body(u_ref, i_ref,
              W1_ref, b1_ref, W2_ref, b2_ref, Wa_ref, ba_ref, out_ref):
  dn = (((1,), (1,)), ((), ()))
  f32 = jnp.float32
  um = u_ref[:, :D]
  im = i_ref[:, :D]
  W1 = W1_ref[...]
  h = (lax.dot_general(um, W1[:, :D], dn, preferred_element_type=f32)
       + lax.dot_general(im, W1[:, D:], dn, preferred_element_type=f32)
       + b1_ref[...])
  h = jnp.maximum(h, 0.0)
  h = lax.dot_general(h, W2_ref[...], dn, preferred_element_type=f32) + b2_ref[...]
  h = jnp.maximum(h, 0.0)
  g = u_ref[:, D:] * i_ref[:, D:]
  Wa = Wa_ref[...]
  out = (lax.dot_general(h, Wa[:, :32], dn, preferred_element_type=f32)
         + lax.dot_general(g, Wa[:, 32:], dn, preferred_element_type=f32)
         + ba_ref[...])
  out_ref[...] = out


_mlp = pl.pallas_call(
    _mlp_body,
    grid=(B // BB,),
    in_specs=[
        pl.BlockSpec((BB, DP), lambda i: (i, 0)),
        pl.BlockSpec((BB, DP), lambda i: (i, 0)),
        pl.BlockSpec((64, 128), lambda i: (0, 0)),
        pl.BlockSpec((1, 64), lambda i: (0, 0)),
        pl.BlockSpec((32, 64), lambda i: (0, 0)),
        pl.BlockSpec((1, 32), lambda i: (0, 0)),
        pl.BlockSpec((1, 96), lambda i: (0, 0)),
        pl.BlockSpec((1, 1), lambda i: (0, 0)),
    ],
    out_specs=pl.BlockSpec((BB, 1), lambda i: (i, 0)),
    out_shape=jax.ShapeDtypeStruct((B, 1), jnp.float32),
)


def kernel(usr_comments, descriptions, emb_user_mlp, emb_item_mlp,
           emb_user_gmf, emb_item_gmf, W1, b1, W2, b2, Wa, ba):
  pad_vals = jnp.arange(B * (LP - L), dtype=jnp.int32).reshape(B, LP - L) % V
  usr_p = jnp.concatenate([usr_comments, pad_vals], axis=1).reshape(-1)
  desc_p = jnp.concatenate([descriptions, pad_vals], axis=1).reshape(-1)
  tu, ti = _fuse(emb_user_mlp.T, emb_user_gmf.T,
                 emb_item_mlp.T, emb_item_gmf.T)
  pu, pi = _pool(usr_p, desc_p, tu, ti)
  return _mlp(pu, pi, W1, b1.reshape(1, -1), W2, b2.reshape(1, -1),
              Wa, ba.reshape(1, 1))
